# Initial kernel scaffold; baseline (speedup 1.0000x reference)
import jax
import jax.numpy as jnp
from jax.experimental import pallas as pl


def kernel(x, edge_index, W1, W2):
    raise NotImplementedError("write your pallas kernel here")


# speedup vs baseline: 4.9445x; 4.9445x over previous
import functools

import jax
import jax.numpy as jnp
import numpy as np
from jax import lax
from jax.experimental import pallas as pl
from jax.experimental.pallas import tpu as pltpu
from jax.experimental.pallas import tpu_sc as plsc

N = 10000
PN = 10240
E = 160000
PE = 163840
F = 256
HF = 128
NSUB = 16
CHUNK = 128
CH_PER_SUB = (PE // NSUB) // CHUNK

_MESH = plsc.VectorSubcoreMesh(
    core_axis_name="c", subcore_axis_name="s", num_cores=2, num_subcores=NSUB
)


@functools.partial(
    pl.kernel,
    out_type=(
        jax.ShapeDtypeStruct((PN, HF), jnp.float32),
        jax.ShapeDtypeStruct((PN, HF), jnp.float32),
    ),
    mesh=_MESH,
    compiler_params=pltpu.CompilerParams(needs_layout_passes=False),
    scratch_types=[
        pltpu.VMEM((1, CHUNK), jnp.int32),
        pltpu.VMEM((CHUNK,), jnp.float32),
        pltpu.VMEM((640,), jnp.float32),
        pltpu.VMEM((640, HF), jnp.float32),
        pltpu.VMEM_SHARED((PN,), jnp.float32),
    ],
)
def _sc_scales(src_hbm, dst_hbm, sout_hbm, sin_hbm, idxb, onesb, cntv, srep, cnt_sp):
    c = lax.axis_index("c")
    s = lax.axis_index("s")

    for j in range(CHUNK // 16):
        onesb[pl.ds(j * 16, 16)] = jnp.full((16,), 1.0, jnp.float32)
    for j in range(640 // 16):
        cntv[pl.ds(j * 16, 16)] = jnp.zeros((16,), jnp.float32)
    pltpu.sync_copy(cntv, cnt_sp.at[pl.ds(s * 640, 640)])
    plsc.subcore_barrier()

    @pl.loop(0, CH_PER_SUB)
    def _count(g):
        base = (s * CH_PER_SUB + g) * CHUNK

        @pl.when(c == 0)
        def _():
            pltpu.sync_copy(src_hbm.at[pl.ds(base, CHUNK)], idxb.at[0])

        @pl.when(c == 1)
        def _():
            pltpu.sync_copy(dst_hbm.at[pl.ds(base, CHUNK)], idxb.at[0])

        pltpu.sync_copy(onesb, cnt_sp.at[idxb.at[0]], add=True)

    plsc.subcore_barrier()

    pltpu.sync_copy(cnt_sp.at[pl.ds(s * 640, 640)], cntv)

    @pl.loop(0, 640)
    def _rep(r):
        val = plsc.load_gather(cntv, [jnp.full((16,), r, jnp.int32)])
        for j in range(HF // 16):
            srep[r, pl.ds(j * 16, 16)] = val

    @pl.when(c == 0)
    def _():
        pltpu.sync_copy(srep, sout_hbm.at[pl.ds(s * 640, 640)])

    @pl.when(c == 1)
    def _():
        pltpu.sync_copy(srep, sin_hbm.at[pl.ds(s * 640, 640)])


@functools.partial(
    pl.kernel,
    out_type=jax.ShapeDtypeStruct((2, PN, HF), jnp.float32),
    mesh=_MESH,
    compiler_params=pltpu.CompilerParams(needs_layout_passes=False),
    scratch_types=[
        pltpu.VMEM((2, CHUNK), jnp.int32),
        pltpu.VMEM((2, CHUNK), jnp.int32),
        pltpu.VMEM((2, CHUNK, HF), jnp.float32),
        pltpu.VMEM_SHARED((PN, HF), jnp.float32),
        pltpu.SemaphoreType.DMA,
        pltpu.SemaphoreType.DMA,
    ],
)
def _sc_agg(table_hbm, src_hbm, dst_hbm, out_hbm, srcb, dstb, rowsb, accum, sem0, sem1):
    c = lax.axis_index("c")
    s = lax.axis_index("s")
    sems = (sem0, sem1)
    ebase = s * (PE // NSUB)

    @pl.loop(0, CHUNK)
    def _z(r):
        for j in range(HF // 16):
            rowsb[0, r, pl.ds(j * 16, 16)] = jnp.zeros((16,), jnp.float32)

    for k in range(640 // CHUNK):
        pltpu.sync_copy(rowsb.at[0], accum.at[pl.ds(s * 640 + k * CHUNK, CHUNK)])
    plsc.subcore_barrier()

    roff = c * PN

    def fetch(g, b):
        base = ebase + g * CHUNK
        pltpu.sync_copy(src_hbm.at[pl.ds(base, CHUNK)], srcb.at[b])
        pltpu.sync_copy(dst_hbm.at[pl.ds(base, CHUNK)], dstb.at[b])
        for j in range(CHUNK // 16):
            srcb[b, pl.ds(j * 16, 16)] = srcb[b, pl.ds(j * 16, 16)] + roff
        pltpu.async_copy(table_hbm.at[srcb.at[b]], rowsb.at[b], sems[b])

    def drain(g, b):
        pltpu.make_async_copy(table_hbm.at[srcb.at[b]], rowsb.at[b], sems[b]).wait()
        pltpu.sync_copy(rowsb.at[b], accum.at[dstb.at[b]], add=True)

    fetch(0, 0)

    @pl.loop(0, CH_PER_SUB - 2, step=2)
    def _main(g):
        fetch(g + 1, 1)
        drain(g, 0)
        fetch(g + 2, 0)
        drain(g + 1, 1)

    fetch(CH_PER_SUB - 1, 1)
    drain(CH_PER_SUB - 2, 0)
    drain(CH_PER_SUB - 1, 1)

    plsc.subcore_barrier()
    pltpu.sync_copy(accum.at[pl.ds(s * 640, 640)], out_hbm.at[c, pl.ds(s * 640, 640)])


def _rsqrt_col(cnt_ref):
    return lax.rsqrt(jnp.maximum(cnt_ref[:, :1], 1.0))


def _prep_body(x_ref, cout_ref, out_ref):
    out_ref[0] = x_ref[...] * _rsqrt_col(cout_ref)


def _mid_body(agg_ref, w1_ref, w2_ref, cin_ref, cout_ref, out_ref):
    a0 = agg_ref[0]
    a1 = agg_ref[1]
    t = jnp.dot(a0, w1_ref[:HF, :], preferred_element_type=jnp.float32)
    t += jnp.dot(a1, w1_ref[HF:, :], preferred_element_type=jnp.float32)
    h1 = jnp.maximum(t * _rsqrt_col(cin_ref), 0.0)
    g = h1 * _rsqrt_col(cout_ref)
    r = jnp.dot(g, w2_ref[...], preferred_element_type=jnp.float32)
    out_ref[0] = r[:, :HF]
    out_ref[1] = r[:, HF:]


def _final_body(agg_ref, cin_ref, out_ref):
    sc = _rsqrt_col(cin_ref)
    out_ref[...] = jnp.concatenate([agg_ref[0] * sc, agg_ref[1] * sc], axis=1)


def kernel(x, edge_index, W1, W2):
    f32 = jnp.float32
    src = edge_index[0]
    dst = edge_index[1]
    pad_idx = jnp.asarray(N + (np.arange(PE - E) % 64), jnp.int32)
    src_p = jnp.concatenate([src, pad_idx])
    dst_p = jnp.concatenate([dst, pad_idx])
    x_p = jnp.pad(x, ((0, PN - N), (0, 0)))

    cout_rep, cin_rep = _sc_scales(src_p, dst_p)

    h0 = pl.pallas_call(
        _prep_body,
        grid=(PN // 512, 2),
        in_specs=[
            pl.BlockSpec((512, HF), lambda r, h: (r, h)),
            pl.BlockSpec((512, HF), lambda r, h: (r, 0)),
        ],
        out_specs=pl.BlockSpec((1, 512, HF), lambda r, h: (h, r, 0)),
        out_shape=jax.ShapeDtypeStruct((2, PN, HF), f32),
    )(x_p, cout_rep)

    a1 = _sc_agg(h0.reshape(2 * PN, HF), src_p, dst_p)

    h2 = pl.pallas_call(
        _mid_body,
        grid=(PN // 512,),
        in_specs=[
            pl.BlockSpec((2, 512, HF), lambda r: (0, r, 0)),
            pl.BlockSpec((F, 512), lambda r: (0, 0)),
            pl.BlockSpec((512, F), lambda r: (0, 0)),
            pl.BlockSpec((512, HF), lambda r: (r, 0)),
            pl.BlockSpec((512, HF), lambda r: (r, 0)),
        ],
        out_specs=pl.BlockSpec((2, 512, HF), lambda r: (0, r, 0)),
        out_shape=jax.ShapeDtypeStruct((2, PN, HF), f32),
    )(a1, W1, W2, cin_rep, cout_rep)

    a2 = _sc_agg(h2.reshape(2 * PN, HF), src_p, dst_p)

    out = pl.pallas_call(
        _final_body,
        grid=(N // 400,),
        in_specs=[
            pl.BlockSpec((2, 400, HF), lambda r: (0, r, 0)),
            pl.BlockSpec((400, HF), lambda r: (r, 0)),
        ],
        out_specs=pl.BlockSpec((400, F), lambda r: (r, 0)),
        out_shape=jax.ShapeDtypeStruct((N, F), f32),
    )(a2, cin_rep)
    return out
